# baseline (speedup 1.0000x reference)
import functools

import jax
import jax.numpy as jnp
from jax import lax
from jax.experimental import pallas as pl
from jax.experimental.pallas import tpu as pltpu
from jax.experimental.pallas import tpu_sc as plsc

N = 10000
E = 320000
IN_DIM = 128
HID = 64
OUT = 32
HAUG = 64

NC = 2
NS = 16
NW = NC * NS
EPW = E // NW
CB = 80
NCHUNK = EPW // CB
ROWS_PT = N // NS


def _tc1_body(x_ref, w1_ref, asp_ref, adp_ref, h_ref, asrc_ref, adst_ref):
    x = x_ref[...]
    h = jnp.dot(x, w1_ref[...], preferred_element_type=jnp.float32)
    h_ref[...] = h
    asrc_ref[...] = jnp.sum(h * asp_ref[...][None, :], axis=1).reshape(1, -1)
    adst_ref[...] = jnp.sum(h * adp_ref[...][None, :], axis=1).reshape(1, -1)


def _tc1(features, w1, asp, adp):
    return pl.pallas_call(
        _tc1_body,
        out_shape=[
            jax.ShapeDtypeStruct((N, HID), jnp.float32),
            jax.ShapeDtypeStruct((1, N), jnp.float32),
            jax.ShapeDtypeStruct((1, N), jnp.float32),
        ],
    )(features, w1, asp, adp)


def _elu(x):
    return jnp.where(x > 0, x, jnp.exp(jnp.minimum(x, 0.0)) - 1.0)


def _tc2_body(p_ref, dp_ref, w2_ref, h2_ref, h3p_ref, den_ref):
    p = p_ref[...]
    num = p[0] + p[1]
    dp = dp_ref[...].reshape(NW, -1)
    den = jnp.sum(dp, axis=0)[:, None] + 1e-16
    h1 = _elu(num / den)
    h2 = jnp.dot(h1, w2_ref[...], preferred_element_type=jnp.float32)
    h2_ref[...] = h2
    h3p_ref[...] = lax.dot_general(h2, w2_ref[...], (((1,), (1,)), ((), ())),
                                   preferred_element_type=jnp.float32)
    den_ref[...] = den.reshape(1, -1)


def _tc2(part1, den_parts, w2):
    return pl.pallas_call(
        _tc2_body,
        out_shape=[
            jax.ShapeDtypeStruct((N, OUT), jnp.float32),
            jax.ShapeDtypeStruct((N, HID), jnp.float32),
            jax.ShapeDtypeStruct((1, N), jnp.float32),
        ],
    )(part1, den_parts, w2)


def _tc3_body(p_ref, den_ref, w1_ref, h4_ref):
    p = p_ref[...]
    s = p[0] + p[1]
    den = den_ref[...].reshape(-1)[:, None]
    h3 = _elu(s / den)
    h4_ref[...] = lax.dot_general(h3, w1_ref[...], (((1,), (1,)), ((), ())),
                                  preferred_element_type=jnp.float32)


def _tc3(part3, den, w1):
    return pl.pallas_call(
        _tc3_body,
        out_shape=[jax.ShapeDtypeStruct((N, IN_DIM), jnp.float32)],
    )(part3, den, w1)


_MESH = plsc.VectorSubcoreMesh(core_axis_name="c", subcore_axis_name="s")
_SC_PARAMS = pltpu.CompilerParams(use_tc_tiling_on_sc=False,
                                  needs_layout_passes=False)


def _zero_vmem(buf):
    zc = jnp.zeros((16,), jnp.float32)
    lanes = buf.shape[1] // 16

    def body(r, c):
        for f in range(lanes):
            buf[r, pl.ds(f * 16, 16)] = zc
        return c

    lax.fori_loop(0, buf.shape[0], body, 0)


def _init_acc(rows5, acc, s):
    zb = rows5.at[0]
    _zero_vmem(zb)
    for k in range(ROWS_PT // CB):
        pltpu.sync_copy(zb, acc.at[pl.ds(s * ROWS_PT + k * CB, CB)])
    pltpu.sync_copy(zb, acc.at[pl.ds(s * ROWS_PT + ROWS_PT - CB, CB)])


def _scale_chunk(rows, e2_v, j):
    lanes = rows.shape[1] // 16

    @plsc.parallel_loop(0, CB // 16, unroll=CB // 16)
    def grp(g):
        off = pl.multiple_of(g * 16, 16)
        ev = e2_v[j, pl.ds(off, 16)]
        base = g * 16
        for l in range(16):
            s = ev[l]
            for f in range(lanes):
                sl = pl.ds(f * 16, 16)
                rows[base + l, sl] = rows[base + l, sl] * s


NB = 5
NRING = (NCHUNK // NB) * NB


def _agg_loop(h_hbm, src2_v, dst2_v, e2_v, rows5, gsem, ssem, acc,
              ew_fn=None):
    def outer(g, c):
        gd = [pltpu.async_copy(h_hbm.at[src2_v.at[g * NB + b]],
                               rows5.at[b], gsem.at[b])
              for b in range(NB)]
        if ew_fn is not None:
            for b in range(NB):
                ew_fn(g * NB + b)
        sd = []
        for b in range(NB):
            j = g * NB + b
            gd[b].wait()
            _scale_chunk(rows5.at[b], e2_v, j)
            sd.append(pltpu.async_copy(rows5.at[b], acc.at[dst2_v.at[j]],
                                       ssem.at[b], add=True))
        for d in sd:
            d.wait()
        return c

    lax.fori_loop(0, NRING // NB, outer, 0)
    for j in range(NRING, NCHUNK):
        pltpu.sync_copy(h_hbm.at[src2_v.at[j]], rows5.at[0])
        if ew_fn is not None:
            ew_fn(j)
        _scale_chunk(rows5.at[0], e2_v, j)
        pltpu.sync_copy(rows5.at[0], acc.at[dst2_v.at[j]], add=True)


def _sc1_body(h_hbm, asrc_hbm, adst_hbm, src3_hbm,
              dst3_hbm, out_hbm, e_hbm, den_hbm,
              asrc_v, adst_v, src2_v, dst2_v, e2_v, den_v, rows5, gsem, ssem,
              acc):
    c = lax.axis_index("c")
    s = lax.axis_index("s")
    w = c * NS + s

    pltpu.sync_copy(asrc_hbm.at[0], asrc_v)
    pltpu.sync_copy(adst_hbm.at[0], adst_v)
    pltpu.sync_copy(src3_hbm.at[w], src2_v)
    pltpu.sync_copy(dst3_hbm.at[w], dst2_v)

    zc = jnp.zeros((16,), jnp.float32)

    def zden(i, c_):
        den_v[pl.ds(i * 16, 16)] = zc
        return c_

    lax.fori_loop(0, N // 16, zden, 0)

    def ew_fn(j):
        for g in range(CB // 16):
            sl = pl.ds(g * 16, 16)
            d16 = dst2_v[j, sl]
            a = (plsc.load_gather(asrc_v, [src2_v[j, sl]])
                 + plsc.load_gather(adst_v, [d16]))
            sig = 1.0 / (1.0 + jnp.exp(-a))
            ev = jnp.exp(sig)
            e2_v[j, sl] = ev
            plsc.addupdate_scatter(den_v, [d16], ev)

    _init_acc(rows5, acc, s)
    plsc.subcore_barrier()

    _agg_loop(h_hbm, src2_v, dst2_v, e2_v, rows5, gsem, ssem, acc, ew_fn)

    pltpu.sync_copy(e2_v, e_hbm.at[w])
    pltpu.sync_copy(den_v, den_hbm.at[c].at[s])
    plsc.subcore_barrier()
    pltpu.sync_copy(acc.at[pl.ds(s * ROWS_PT, ROWS_PT)],
                    out_hbm.at[c].at[pl.ds(s * ROWS_PT, ROWS_PT)])


def _sc2_body(h_hbm, src3_hbm, dst3_hbm, e_all_hbm,
              out_hbm,
              src2_v, dst2_v, e2_v, rows5, gsem, ssem, acc):
    c = lax.axis_index("c")
    s = lax.axis_index("s")
    w = c * NS + s

    pltpu.sync_copy(src3_hbm.at[w], src2_v)
    pltpu.sync_copy(dst3_hbm.at[w], dst2_v)
    pltpu.sync_copy(e_all_hbm.at[w], e2_v)

    _init_acc(rows5, acc, s)
    plsc.subcore_barrier()

    _agg_loop(h_hbm, src2_v, dst2_v, e2_v, rows5, gsem, ssem, acc)

    plsc.subcore_barrier()
    pltpu.sync_copy(acc.at[pl.ds(s * ROWS_PT, ROWS_PT)],
                    out_hbm.at[c].at[pl.ds(s * ROWS_PT, ROWS_PT)])


_sc1 = functools.partial(
    pl.kernel,
    out_type=(
        jax.ShapeDtypeStruct((NC, N, HAUG), jnp.float32),
        jax.ShapeDtypeStruct((NW, NCHUNK, CB), jnp.float32),
        jax.ShapeDtypeStruct((NC, NS, N), jnp.float32),
    ),
    mesh=_MESH,
    scratch_types=[
        pltpu.VMEM((N,), jnp.float32),
        pltpu.VMEM((N,), jnp.float32),
        pltpu.VMEM((NCHUNK, CB), jnp.int32),
        pltpu.VMEM((NCHUNK, CB), jnp.int32),
        pltpu.VMEM((NCHUNK, CB), jnp.float32),
        pltpu.VMEM((N,), jnp.float32),
        pltpu.VMEM((NB, CB, HAUG), jnp.float32),
        pltpu.SemaphoreType.DMA((NB,)),
        pltpu.SemaphoreType.DMA((NB,)),
        pltpu.VMEM_SHARED((N, HAUG), jnp.float32),
    ],
    compiler_params=_SC_PARAMS,
)(_sc1_body)


_sc2 = functools.partial(
    pl.kernel,
    out_type=jax.ShapeDtypeStruct((NC, N, HID), jnp.float32),
    mesh=_MESH,
    scratch_types=[
        pltpu.VMEM((NCHUNK, CB), jnp.int32),
        pltpu.VMEM((NCHUNK, CB), jnp.int32),
        pltpu.VMEM((NCHUNK, CB), jnp.float32),
        pltpu.VMEM((NB, CB, HID), jnp.float32),
        pltpu.SemaphoreType.DMA((NB,)),
        pltpu.SemaphoreType.DMA((NB,)),
        pltpu.VMEM_SHARED((N, HID), jnp.float32),
    ],
    compiler_params=_SC_PARAMS,
)(_sc2_body)


def kernel(features, edge_index, W1, W2, att_src1, att_dst1):
    src = edge_index[0].astype(jnp.int32)
    dst = edge_index[1].astype(jnp.int32)
    src3 = src.reshape(NW, NCHUNK, CB)
    dst3 = dst.reshape(NW, NCHUNK, CB)

    h1_pre, asrc, adst = _tc1(features, W1, att_src1, att_dst1)
    part1, e_all, den_parts = _sc1(h1_pre, asrc, adst, src3, dst3)
    h2, h3_pre, den = _tc2(part1, den_parts, W2)
    part3 = _sc2(h3_pre, src3, dst3, e_all)
    (h4,) = _tc3(part3, den, W1)
    return (h2, h4)


# speedup vs baseline: 1.0775x; 1.0775x over previous
import functools

import jax
import jax.numpy as jnp
from jax import lax
from jax.experimental import pallas as pl
from jax.experimental.pallas import tpu as pltpu
from jax.experimental.pallas import tpu_sc as plsc

N = 10000
E = 320000
IN_DIM = 128
HID = 64
OUT = 32
HAUG = 64

NC = 2
NS = 16
NW = NC * NS
EPW = E // NW
CB = 80
NCHUNK = EPW // CB
ROWS_PT = N // NS


def _tc1_body(x_ref, w1_ref, asp_ref, adp_ref, h_ref, asrc_ref, adst_ref):
    x = x_ref[...]
    h = jnp.dot(x, w1_ref[...], preferred_element_type=jnp.float32)
    h_ref[...] = h
    asrc_ref[...] = jnp.sum(h * asp_ref[...][None, :], axis=1).reshape(1, -1)
    adst_ref[...] = jnp.sum(h * adp_ref[...][None, :], axis=1).reshape(1, -1)


def _tc1(features, w1, asp, adp):
    return pl.pallas_call(
        _tc1_body,
        out_shape=[
            jax.ShapeDtypeStruct((N, HID), jnp.float32),
            jax.ShapeDtypeStruct((1, N), jnp.float32),
            jax.ShapeDtypeStruct((1, N), jnp.float32),
        ],
    )(features, w1, asp, adp)


def _elu(x):
    return jnp.where(x > 0, x, jnp.exp(jnp.minimum(x, 0.0)) - 1.0)


def _tc2_body(p_ref, dp_ref, w2_ref, h2_ref, h3p_ref, den_ref):
    p = p_ref[...]
    num = p[0] + p[1]
    dp = dp_ref[...].reshape(NW, -1)
    den = jnp.sum(dp, axis=0)[:, None] + 1e-16
    h1 = _elu(num / den)
    h2 = jnp.dot(h1, w2_ref[...], preferred_element_type=jnp.float32)
    h2_ref[...] = h2
    h3p_ref[...] = lax.dot_general(h2, w2_ref[...], (((1,), (1,)), ((), ())),
                                   preferred_element_type=jnp.float32)
    den_ref[...] = den.reshape(1, -1)


def _tc2(part1, den_parts, w2):
    return pl.pallas_call(
        _tc2_body,
        out_shape=[
            jax.ShapeDtypeStruct((N, OUT), jnp.float32),
            jax.ShapeDtypeStruct((N, HID), jnp.float32),
            jax.ShapeDtypeStruct((1, N), jnp.float32),
        ],
    )(part1, den_parts, w2)


def _tc3_body(p_ref, den_ref, w1_ref, h4_ref):
    p = p_ref[...]
    s = p[0] + p[1]
    den = den_ref[...].reshape(-1)[:, None]
    h3 = _elu(s / den)
    h4_ref[...] = lax.dot_general(h3, w1_ref[...], (((1,), (1,)), ((), ())),
                                  preferred_element_type=jnp.float32)


def _tc3(part3, den, w1):
    return pl.pallas_call(
        _tc3_body,
        out_shape=[jax.ShapeDtypeStruct((N, IN_DIM), jnp.float32)],
    )(part3, den, w1)


_MESH = plsc.VectorSubcoreMesh(core_axis_name="c", subcore_axis_name="s")
_SC_PARAMS = pltpu.CompilerParams(use_tc_tiling_on_sc=False,
                                  needs_layout_passes=False)


def _zero_vmem(buf):
    zc = jnp.zeros((16,), jnp.float32)
    lanes = buf.shape[1] // 16

    def body(r, c):
        for f in range(lanes):
            buf[r, pl.ds(f * 16, 16)] = zc
        return c

    lax.fori_loop(0, buf.shape[0], body, 0)


def _init_acc(rows5, acc, s, sems):
    zb = rows5.at[0]
    _zero_vmem(zb)
    ds_ = [pltpu.async_copy(zb, acc.at[pl.ds(s * ROWS_PT + k * CB, CB)],
                            sems.at[k % NB])
           for k in range(ROWS_PT // CB)]
    ds_.append(pltpu.async_copy(
        zb, acc.at[pl.ds(s * ROWS_PT + ROWS_PT - CB, CB)],
        sems.at[(ROWS_PT // CB) % NB]))
    return ds_


def _scale_chunk(rows, e2_v, j):
    lanes = rows.shape[1] // 16

    @plsc.parallel_loop(0, CB // 16, unroll=CB // 16)
    def grp(g):
        off = pl.multiple_of(g * 16, 16)
        ev = e2_v[j, pl.ds(off, 16)]
        base = g * 16
        for l in range(16):
            s = ev[l]
            for f in range(lanes):
                sl = pl.ds(f * 16, 16)
                rows[base + l, sl] = rows[base + l, sl] * s


NB = 4
NRING = (NCHUNK // NB) * NB


def _agg_loop(h_hbm, src2_v, dst2_v, e2_v, rows5, gsem, ssem, acc,
              ew_fn=None):
    def outer(g, c):
        gd = [pltpu.async_copy(h_hbm.at[src2_v.at[g * NB + b]],
                               rows5.at[b], gsem.at[b])
              for b in range(NB)]
        if ew_fn is not None:
            for b in range(NB):
                ew_fn(g * NB + b)
        sd = []
        for b in range(NB):
            j = g * NB + b
            gd[b].wait()
            _scale_chunk(rows5.at[b], e2_v, j)
            sd.append(pltpu.async_copy(rows5.at[b], acc.at[dst2_v.at[j]],
                                       ssem.at[b], add=True))
        for d in sd:
            d.wait()
        return c

    lax.fori_loop(0, NRING // NB, outer, 0)
    for j in range(NRING, NCHUNK):
        pltpu.sync_copy(h_hbm.at[src2_v.at[j]], rows5.at[0])
        if ew_fn is not None:
            ew_fn(j)
        _scale_chunk(rows5.at[0], e2_v, j)
        pltpu.sync_copy(rows5.at[0], acc.at[dst2_v.at[j]], add=True)


def _sc1_body(h_hbm, asrc_hbm, adst_hbm, src3_hbm,
              dst3_hbm, out_hbm, e_hbm, den_hbm,
              asrc_v, adst_v, src2_v, dst2_v, e2_v, den_v, rows5, gsem, ssem,
              acc):
    c = lax.axis_index("c")
    s = lax.axis_index("s")
    w = c * NS + s

    ind = [pltpu.async_copy(asrc_hbm.at[0], asrc_v, gsem.at[0]),
           pltpu.async_copy(adst_hbm.at[0], adst_v, gsem.at[1]),
           pltpu.async_copy(src3_hbm.at[w], src2_v, gsem.at[2]),
           pltpu.async_copy(dst3_hbm.at[w], dst2_v, gsem.at[3])]

    zc = jnp.zeros((16,), jnp.float32)

    def zden(i, c_):
        den_v[pl.ds(i * 16, 16)] = zc
        return c_

    lax.fori_loop(0, N // 16, zden, 0)

    def ew_fn(j):
        for g in range(CB // 16):
            sl = pl.ds(g * 16, 16)
            d16 = dst2_v[j, sl]
            a = (plsc.load_gather(asrc_v, [src2_v[j, sl]])
                 + plsc.load_gather(adst_v, [d16]))
            sig = 1.0 / (1.0 + jnp.exp(-a))
            ev = jnp.exp(sig)
            e2_v[j, sl] = ev
            plsc.addupdate_scatter(den_v, [d16], ev)

    zds = _init_acc(rows5, acc, s, ssem)
    for d in ind + zds:
        d.wait()
    plsc.subcore_barrier()

    _agg_loop(h_hbm, src2_v, dst2_v, e2_v, rows5, gsem, ssem, acc, ew_fn)

    pltpu.sync_copy(e2_v, e_hbm.at[w])
    pltpu.sync_copy(den_v, den_hbm.at[c].at[s])
    plsc.subcore_barrier()
    pltpu.sync_copy(acc.at[pl.ds(s * ROWS_PT, ROWS_PT)],
                    out_hbm.at[c].at[pl.ds(s * ROWS_PT, ROWS_PT)])


def _sc2_body(h_hbm, src3_hbm, dst3_hbm, e_all_hbm,
              out_hbm,
              src2_v, dst2_v, e2_v, rows5, gsem, ssem, acc):
    c = lax.axis_index("c")
    s = lax.axis_index("s")
    w = c * NS + s

    ind = [pltpu.async_copy(src3_hbm.at[w], src2_v, gsem.at[0]),
           pltpu.async_copy(dst3_hbm.at[w], dst2_v, gsem.at[1]),
           pltpu.async_copy(e_all_hbm.at[w], e2_v, gsem.at[2])]
    zds = _init_acc(rows5, acc, s, ssem)
    for d in ind + zds:
        d.wait()
    plsc.subcore_barrier()

    _agg_loop(h_hbm, src2_v, dst2_v, e2_v, rows5, gsem, ssem, acc)

    plsc.subcore_barrier()
    pltpu.sync_copy(acc.at[pl.ds(s * ROWS_PT, ROWS_PT)],
                    out_hbm.at[c].at[pl.ds(s * ROWS_PT, ROWS_PT)])


_sc1 = functools.partial(
    pl.kernel,
    out_type=(
        jax.ShapeDtypeStruct((NC, N, HAUG), jnp.float32),
        jax.ShapeDtypeStruct((NW, NCHUNK, CB), jnp.float32),
        jax.ShapeDtypeStruct((NC, NS, N), jnp.float32),
    ),
    mesh=_MESH,
    scratch_types=[
        pltpu.VMEM((N,), jnp.float32),
        pltpu.VMEM((N,), jnp.float32),
        pltpu.VMEM((NCHUNK, CB), jnp.int32),
        pltpu.VMEM((NCHUNK, CB), jnp.int32),
        pltpu.VMEM((NCHUNK, CB), jnp.float32),
        pltpu.VMEM((N,), jnp.float32),
        pltpu.VMEM((NB, CB, HAUG), jnp.float32),
        pltpu.SemaphoreType.DMA((NB,)),
        pltpu.SemaphoreType.DMA((NB,)),
        pltpu.VMEM_SHARED((N, HAUG), jnp.float32),
    ],
    compiler_params=_SC_PARAMS,
)(_sc1_body)


_sc2 = functools.partial(
    pl.kernel,
    out_type=jax.ShapeDtypeStruct((NC, N, HID), jnp.float32),
    mesh=_MESH,
    scratch_types=[
        pltpu.VMEM((NCHUNK, CB), jnp.int32),
        pltpu.VMEM((NCHUNK, CB), jnp.int32),
        pltpu.VMEM((NCHUNK, CB), jnp.float32),
        pltpu.VMEM((NB, CB, HID), jnp.float32),
        pltpu.SemaphoreType.DMA((NB,)),
        pltpu.SemaphoreType.DMA((NB,)),
        pltpu.VMEM_SHARED((N, HID), jnp.float32),
    ],
    compiler_params=_SC_PARAMS,
)(_sc2_body)


def kernel(features, edge_index, W1, W2, att_src1, att_dst1):
    src = edge_index[0].astype(jnp.int32)
    dst = edge_index[1].astype(jnp.int32)
    src3 = src.reshape(NW, NCHUNK, CB)
    dst3 = dst.reshape(NW, NCHUNK, CB)

    h1_pre, asrc, adst = _tc1(features, W1, att_src1, att_dst1)
    part1, e_all, den_parts = _sc1(h1_pre, asrc, adst, src3, dst3)
    h2, h3_pre, den = _tc2(part1, den_parts, W2)
    part3 = _sc2(h3_pre, src3, dst3, e_all)
    (h4,) = _tc3(part3, den, W1)
    return (h2, h4)
